# baseline (speedup 1.0000x reference)
import functools

import jax
import jax.numpy as jnp
from jax import lax
from jax.experimental import pallas as pl
from jax.experimental.pallas import tpu as pltpu
from jax.experimental.pallas import tpu_sc as plsc

B, S, H = 64, 1024, 1024
EPS = 1e-12

_NC = 2
_NS = 16
_NW = _NC * _NS
_W = H // 4
_CH = 64

_P = 2
_PB = B // _P
_PROWS = _PB * S
_PRPW = _PROWS // _NW
_PNCH = _PRPW // _CH


def _sc_gather(table, idx_piece):
    mesh = plsc.VectorSubcoreMesh(core_axis_name="c", subcore_axis_name="s")

    @functools.partial(
        pl.kernel,
        out_type=jax.ShapeDtypeStruct((_PROWS, _W), jnp.int32),
        mesh=mesh,
        scratch_types=[
            pltpu.VMEM((_PRPW,), jnp.int32),
            pltpu.VMEM((_CH, _W), jnp.int32),
            pltpu.VMEM((_CH, _W), jnp.int32),
            pltpu.SemaphoreType.DMA,
            pltpu.SemaphoreType.DMA,
        ],
    )
    def k(table_hbm, idx_hbm, out_hbm, idx_v, buf0, buf1, sem0, sem1):
        wid = lax.axis_index("s") * _NC + lax.axis_index("c")
        base = wid * _PRPW
        pltpu.sync_copy(idx_hbm.at[pl.ds(base, _PRPW)], idx_v)
        pltpu.async_copy(table_hbm.at[idx_v.at[pl.ds(0, _CH)]], buf0, sem0)

        def step(c, cur, cur_sem, nxt, nxt_sem):
            @pl.when(c + 1 < _PNCH)
            def _():
                pltpu.async_copy(
                    table_hbm.at[idx_v.at[pl.ds((c + 1) * _CH, _CH)]],
                    nxt, nxt_sem,
                )
            pltpu.make_async_copy(
                table_hbm.at[idx_v.at[pl.ds(c * _CH, _CH)]], cur, cur_sem
            ).wait()
            pltpu.sync_copy(cur, out_hbm.at[pl.ds(base + c * _CH, _CH)])

        def body(c, carry):
            @pl.when(c % 2 == 0)
            def _():
                step(c, buf0, sem0, buf1, sem1)

            @pl.when(c % 2 == 1)
            def _():
                step(c, buf1, sem1, buf0, sem0)

            return carry

        lax.fori_loop(0, _PNCH, body, 0)

    return k(table, idx_piece)


_R = 256


def _ln_body(s_ref, x_ref, t_ref, p_ref, g_ref, b_ref, o_ref):
    scale = s_ref[0]
    w = t_ref[...]
    b0 = (w << 24) >> 24
    b1 = (w << 16) >> 24
    b2 = (w << 8) >> 24
    b3 = w >> 24
    t = jnp.concatenate([b0, b1, b2, b3], axis=-1).astype(jnp.float32) * scale
    e = x_ref[...] + t + p_ref[...][None]
    mean = jnp.mean(e, axis=-1, keepdims=True)
    var = jnp.mean(jnp.square(e - mean), axis=-1, keepdims=True)
    xhat = (e - mean) * lax.rsqrt(var + EPS)
    o_ref[...] = xhat * g_ref[...] + b_ref[...]


def _tc_piece(prev, scale, concat, tok_p, pos, gamma, beta, p):
    grid = (S // _R, _PB)
    data_specs = [
        pl.BlockSpec(memory_space=pltpu.MemorySpace.SMEM),
        pl.BlockSpec((1, _R, H), lambda j, b: (p * _PB + b, j, 0)),
        pl.BlockSpec((1, _R, _W), lambda j, b: (b, j, 0)),
        pl.BlockSpec((_R, H), lambda j, b: (j, 0)),
        pl.BlockSpec((1, H), lambda j, b: (0, 0)),
        pl.BlockSpec((1, H), lambda j, b: (0, 0)),
    ]
    out_spec = pl.BlockSpec((1, _R, H), lambda j, b: (p * _PB + b, j, 0))
    out_shape = jax.ShapeDtypeStruct((B, S, H), jnp.float32)
    if prev is None:
        return pl.pallas_call(
            _ln_body,
            grid=grid,
            in_specs=data_specs,
            out_specs=out_spec,
            out_shape=out_shape,
        )(scale, concat, tok_p, pos, gamma, beta)

    def body(prev_ref, *refs):
        _ln_body(*refs)

    return pl.pallas_call(
        body,
        grid=grid,
        in_specs=[pl.BlockSpec(memory_space=pltpu.MemorySpace.HBM)] + data_specs,
        out_specs=out_spec,
        out_shape=out_shape,
        input_output_aliases={0: 0},
    )(prev, scale, concat, tok_p, pos, gamma, beta)


def kernel(concat_embeddings, concat_type, pos_table, tok_table, ln_gamma, ln_beta):
    idx = concat_type.reshape(_P, _PROWS).astype(jnp.int32)
    absmax = jnp.maximum(jnp.max(jnp.abs(tok_table)), 1e-30)
    q = jnp.round(tok_table * (127.0 / absmax)).astype(jnp.int8)
    table_packed = lax.bitcast_convert_type(
        q.reshape(H, 4, _W).transpose(0, 2, 1), jnp.int32
    )
    scale = (absmax / 127.0).reshape(1)
    gamma = ln_gamma.reshape(1, H)
    beta = ln_beta.reshape(1, H)
    toks = [
        _sc_gather(table_packed, idx[p]).reshape(_PB, S, _W) for p in range(_P)
    ]
    out = None
    for p in range(_P):
        out = _tc_piece(
            out, scale, concat_embeddings, toks[p], pos_table, gamma, beta, p
        )
    return out


# speedup vs baseline: 1.3395x; 1.3395x over previous
import functools

import jax
import jax.numpy as jnp
from jax import lax
from jax.experimental import pallas as pl
from jax.experimental.pallas import tpu as pltpu
from jax.experimental.pallas import tpu_sc as plsc

B, S, H = 64, 1024, 1024
EPS = 1e-12

_NC = 2
_NS = 16
_NW = _NC * _NS
_W = H // 4
_CH = 64

_P = 2
_PB = B // _P
_PROWS = _PB * S
_PRPW = _PROWS // _NW
_PNCH = _PRPW // _CH


def _sc_gather(table, idx_piece):
    mesh = plsc.VectorSubcoreMesh(core_axis_name="c", subcore_axis_name="s")

    @functools.partial(
        pl.kernel,
        out_type=jax.ShapeDtypeStruct((_PROWS, _W), jnp.int32),
        mesh=mesh,
        scratch_types=[
            pltpu.VMEM((_PRPW,), jnp.int32),
            pltpu.VMEM((_CH, _W), jnp.int32),
            pltpu.VMEM((_CH, _W), jnp.int32),
            pltpu.SemaphoreType.DMA,
            pltpu.SemaphoreType.DMA,
        ],
    )
    def k(table_hbm, idx_hbm, out_hbm, idx_v, buf0, buf1, sem0, sem1):
        wid = lax.axis_index("s") * _NC + lax.axis_index("c")
        base = wid * _PRPW
        pltpu.sync_copy(idx_hbm.at[pl.ds(base, _PRPW)], idx_v)
        pltpu.async_copy(table_hbm.at[idx_v.at[pl.ds(0, _CH)]], buf0, sem0)

        def step(c, cur, cur_sem, nxt, nxt_sem):
            @pl.when(c + 1 < _PNCH)
            def _():
                pltpu.async_copy(
                    table_hbm.at[idx_v.at[pl.ds((c + 1) * _CH, _CH)]],
                    nxt, nxt_sem,
                )
            pltpu.make_async_copy(
                table_hbm.at[idx_v.at[pl.ds(c * _CH, _CH)]], cur, cur_sem
            ).wait()
            pltpu.sync_copy(cur, out_hbm.at[pl.ds(base + c * _CH, _CH)])

        def body(c, carry):
            @pl.when(c % 2 == 0)
            def _():
                step(c, buf0, sem0, buf1, sem1)

            @pl.when(c % 2 == 1)
            def _():
                step(c, buf1, sem1, buf0, sem0)

            return carry

        lax.fori_loop(0, _PNCH, body, 0)

    return k(table, idx_piece)


_R = 1024


def _ln_body(s_ref, x_ref, t_ref, p_ref, g_ref, b_ref, o_ref):
    scale = s_ref[0]
    w = t_ref[...]
    b0 = (w << 24) >> 24
    b1 = (w << 16) >> 24
    b2 = (w << 8) >> 24
    b3 = w >> 24
    t = jnp.concatenate([b0, b1, b2, b3], axis=-1).astype(jnp.float32) * scale
    e = x_ref[...] + t + p_ref[...][None]
    mean = jnp.mean(e, axis=-1, keepdims=True)
    var = jnp.mean(jnp.square(e - mean), axis=-1, keepdims=True)
    xhat = (e - mean) * lax.rsqrt(var + EPS)
    o_ref[...] = xhat * g_ref[...] + b_ref[...]


def _tc_piece(prev, scale, concat, tok_p, pos, gamma, beta, p):
    grid = (S // _R, _PB)
    data_specs = [
        pl.BlockSpec(memory_space=pltpu.MemorySpace.SMEM),
        pl.BlockSpec((1, _R, H), lambda j, b: (p * _PB + b, j, 0)),
        pl.BlockSpec((1, _R, _W), lambda j, b: (b, j, 0)),
        pl.BlockSpec((_R, H), lambda j, b: (j, 0)),
        pl.BlockSpec((1, H), lambda j, b: (0, 0)),
        pl.BlockSpec((1, H), lambda j, b: (0, 0)),
    ]
    out_spec = pl.BlockSpec((1, _R, H), lambda j, b: (p * _PB + b, j, 0))
    out_shape = jax.ShapeDtypeStruct((B, S, H), jnp.float32)
    if prev is None:
        return pl.pallas_call(
            _ln_body,
            grid=grid,
            in_specs=data_specs,
            out_specs=out_spec,
            out_shape=out_shape,
        )(scale, concat, tok_p, pos, gamma, beta)

    def body(prev_ref, *refs):
        _ln_body(*refs)

    return pl.pallas_call(
        body,
        grid=grid,
        in_specs=[pl.BlockSpec(memory_space=pltpu.MemorySpace.HBM)] + data_specs,
        out_specs=out_spec,
        out_shape=out_shape,
        input_output_aliases={0: 0},
    )(prev, scale, concat, tok_p, pos, gamma, beta)


def kernel(concat_embeddings, concat_type, pos_table, tok_table, ln_gamma, ln_beta):
    idx = concat_type.reshape(_P, _PROWS).astype(jnp.int32)
    absmax = jnp.maximum(jnp.max(jnp.abs(tok_table)), 1e-30)
    q = jnp.round(tok_table * (127.0 / absmax)).astype(jnp.int8)
    table_packed = lax.bitcast_convert_type(
        q.reshape(H, 4, _W).transpose(0, 2, 1), jnp.int32
    )
    scale = (absmax / 127.0).reshape(1)
    gamma = ln_gamma.reshape(1, H)
    beta = ln_beta.reshape(1, H)
    toks = [
        _sc_gather(table_packed, idx[p]).reshape(_PB, S, _W) for p in range(_P)
    ]
    out = None
    for p in range(_P):
        out = _tc_piece(
            out, scale, concat_embeddings, toks[p], pos_table, gamma, beta, p
        )
    return out


# speedup vs baseline: 1.3797x; 1.0300x over previous
import functools

import jax
import jax.numpy as jnp
from jax import lax
from jax.experimental import pallas as pl
from jax.experimental.pallas import tpu as pltpu
from jax.experimental.pallas import tpu_sc as plsc

B, S, H = 64, 1024, 1024
EPS = 1e-12

_NC = 2
_NS = 16
_NW = _NC * _NS
_W = H // 4
_CH = 64

_P = 2
_PB = B // _P
_PROWS = _PB * S
_PRPW = _PROWS // _NW
_PNCH = _PRPW // _CH


def _sc_gather(table, idx_piece):
    mesh = plsc.VectorSubcoreMesh(core_axis_name="c", subcore_axis_name="s")

    @functools.partial(
        pl.kernel,
        out_type=jax.ShapeDtypeStruct((_PROWS, _W), jnp.int32),
        mesh=mesh,
        scratch_types=[
            pltpu.VMEM((_PRPW,), jnp.int32),
            pltpu.VMEM((_CH, _W), jnp.int32),
            pltpu.VMEM((_CH, _W), jnp.int32),
            pltpu.SemaphoreType.DMA,
            pltpu.SemaphoreType.DMA,
        ],
    )
    def k(table_hbm, idx_hbm, out_hbm, idx_v, buf0, buf1, sem0, sem1):
        wid = lax.axis_index("s") * _NC + lax.axis_index("c")
        base = wid * _PRPW
        pltpu.sync_copy(idx_hbm.at[pl.ds(base, _PRPW)], idx_v)
        pltpu.async_copy(table_hbm.at[idx_v.at[pl.ds(0, _CH)]], buf0, sem0)

        def step(c, cur, cur_sem, nxt, nxt_sem):
            @pl.when(c + 1 < _PNCH)
            def _():
                pltpu.async_copy(
                    table_hbm.at[idx_v.at[pl.ds((c + 1) * _CH, _CH)]],
                    nxt, nxt_sem,
                )
            pltpu.make_async_copy(
                table_hbm.at[idx_v.at[pl.ds(c * _CH, _CH)]], cur, cur_sem
            ).wait()
            pltpu.sync_copy(cur, out_hbm.at[pl.ds(base + c * _CH, _CH)])

        def body(c, carry):
            @pl.when(c % 2 == 0)
            def _():
                step(c, buf0, sem0, buf1, sem1)

            @pl.when(c % 2 == 1)
            def _():
                step(c, buf1, sem1, buf0, sem0)

            return carry

        lax.fori_loop(0, _PNCH, body, 0)

    return k(table, idx_piece)


_R = 1024
_BB = 2


def _ln_body(s_ref, x_ref, t_ref, p_ref, g_ref, b_ref, o_ref):
    scale = s_ref[0]
    w = t_ref[...]
    b0 = (w << 24) >> 24
    b1 = (w << 16) >> 24
    b2 = (w << 8) >> 24
    b3 = w >> 24
    t = jnp.concatenate([b0, b1, b2, b3], axis=-1).astype(jnp.float32) * scale
    e = x_ref[...] + t + p_ref[...][None]
    mean = jnp.mean(e, axis=-1, keepdims=True)
    var = jnp.mean(jnp.square(e - mean), axis=-1, keepdims=True)
    xhat = (e - mean) * lax.rsqrt(var + EPS)
    o_ref[...] = xhat * g_ref[...] + b_ref[...]


def _tc_piece(prev, scale, concat, tok_p, pos, gamma, beta, p):
    grid = (S // _R, _PB // _BB)
    data_specs = [
        pl.BlockSpec(memory_space=pltpu.MemorySpace.SMEM),
        pl.BlockSpec((_BB, _R, H), lambda j, b: (p * _PB // _BB + b, j, 0)),
        pl.BlockSpec((_BB, _R, _W), lambda j, b: (b, j, 0)),
        pl.BlockSpec((_R, H), lambda j, b: (j, 0)),
        pl.BlockSpec((1, H), lambda j, b: (0, 0)),
        pl.BlockSpec((1, H), lambda j, b: (0, 0)),
    ]
    out_spec = pl.BlockSpec((_BB, _R, H), lambda j, b: (p * _PB // _BB + b, j, 0))
    out_shape = jax.ShapeDtypeStruct((B, S, H), jnp.float32)
    if prev is None:
        return pl.pallas_call(
            _ln_body,
            grid=grid,
            in_specs=data_specs,
            out_specs=out_spec,
            out_shape=out_shape,
        )(scale, concat, tok_p, pos, gamma, beta)

    def body(prev_ref, *refs):
        _ln_body(*refs)

    return pl.pallas_call(
        body,
        grid=grid,
        in_specs=[pl.BlockSpec(memory_space=pltpu.MemorySpace.HBM)] + data_specs,
        out_specs=out_spec,
        out_shape=out_shape,
        input_output_aliases={0: 0},
    )(prev, scale, concat, tok_p, pos, gamma, beta)


def kernel(concat_embeddings, concat_type, pos_table, tok_table, ln_gamma, ln_beta):
    idx = concat_type.reshape(_P, _PROWS).astype(jnp.int32)
    absmax = jnp.maximum(jnp.max(jnp.abs(tok_table)), 1e-30)
    q = jnp.round(tok_table * (127.0 / absmax)).astype(jnp.int8)
    table_packed = lax.bitcast_convert_type(
        q.reshape(H, 4, _W).transpose(0, 2, 1), jnp.int32
    )
    scale = (absmax / 127.0).reshape(1)
    gamma = ln_gamma.reshape(1, H)
    beta = ln_beta.reshape(1, H)
    toks = [
        _sc_gather(table_packed, idx[p]).reshape(_PB, S, _W) for p in range(_P)
    ]
    out = None
    for p in range(_P):
        out = _tc_piece(
            out, scale, concat_embeddings, toks[p], pos_table, gamma, beta, p
        )
    return out


# speedup vs baseline: 1.3817x; 1.0014x over previous
import functools

import jax
import jax.numpy as jnp
from jax import lax
from jax.experimental import pallas as pl
from jax.experimental.pallas import tpu as pltpu
from jax.experimental.pallas import tpu_sc as plsc

B, S, H = 64, 1024, 1024
EPS = 1e-12

_NC = 2
_NS = 16
_NW = _NC * _NS
_W = H // 4
_CH = 64

_P = 2
_PB = B // _P
_PROWS = _PB * S
_PRPW = _PROWS // _NW
_PNCH = _PRPW // _CH


def _sc_gather(table, idx_piece):
    mesh = plsc.VectorSubcoreMesh(core_axis_name="c", subcore_axis_name="s")

    @functools.partial(
        pl.kernel,
        out_type=jax.ShapeDtypeStruct((_PROWS, _W), jnp.int32),
        mesh=mesh,
        scratch_types=[
            pltpu.VMEM((_PRPW,), jnp.int32),
            pltpu.VMEM((_CH, _W), jnp.int32),
            pltpu.VMEM((_CH, _W), jnp.int32),
            pltpu.SemaphoreType.DMA,
            pltpu.SemaphoreType.DMA,
        ],
    )
    def k(table_hbm, idx_hbm, out_hbm, idx_v, buf0, buf1, sem0, sem1):
        wid = lax.axis_index("s") * _NC + lax.axis_index("c")
        base = wid * _PRPW
        pltpu.sync_copy(idx_hbm.at[pl.ds(base, _PRPW)], idx_v)
        pltpu.async_copy(table_hbm.at[idx_v.at[pl.ds(0, _CH)]], buf0, sem0)

        def step(c, cur, cur_sem, nxt, nxt_sem):
            @pl.when(c + 1 < _PNCH)
            def _():
                pltpu.async_copy(
                    table_hbm.at[idx_v.at[pl.ds((c + 1) * _CH, _CH)]],
                    nxt, nxt_sem,
                )
            pltpu.make_async_copy(
                table_hbm.at[idx_v.at[pl.ds(c * _CH, _CH)]], cur, cur_sem
            ).wait()
            pltpu.sync_copy(cur, out_hbm.at[pl.ds(base + c * _CH, _CH)])

        def body(c, carry):
            @pl.when(c % 2 == 0)
            def _():
                step(c, buf0, sem0, buf1, sem1)

            @pl.when(c % 2 == 1)
            def _():
                step(c, buf1, sem1, buf0, sem0)

            return carry

        lax.fori_loop(0, _PNCH, body, 0)

    return k(table, idx_piece)


_R = 512
_BB = 4


def _ln_body(s_ref, x_ref, t_ref, p_ref, g_ref, b_ref, o_ref):
    scale = s_ref[0]
    w = t_ref[...]
    b0 = (w << 24) >> 24
    b1 = (w << 16) >> 24
    b2 = (w << 8) >> 24
    b3 = w >> 24
    t = jnp.concatenate([b0, b1, b2, b3], axis=-1).astype(jnp.float32) * scale
    e = x_ref[...] + t + p_ref[...][None]
    mean = jnp.mean(e, axis=-1, keepdims=True)
    var = jnp.mean(jnp.square(e - mean), axis=-1, keepdims=True)
    xhat = (e - mean) * lax.rsqrt(var + EPS)
    o_ref[...] = xhat * g_ref[...] + b_ref[...]


def _tc_piece(prev, scale, concat, tok_p, pos, gamma, beta, p):
    grid = (S // _R, _PB // _BB)
    data_specs = [
        pl.BlockSpec(memory_space=pltpu.MemorySpace.SMEM),
        pl.BlockSpec((_BB, _R, H), lambda j, b: (p * _PB // _BB + b, j, 0)),
        pl.BlockSpec((_BB, _R, _W), lambda j, b: (b, j, 0)),
        pl.BlockSpec((_R, H), lambda j, b: (j, 0)),
        pl.BlockSpec((1, H), lambda j, b: (0, 0)),
        pl.BlockSpec((1, H), lambda j, b: (0, 0)),
    ]
    out_spec = pl.BlockSpec((_BB, _R, H), lambda j, b: (p * _PB // _BB + b, j, 0))
    out_shape = jax.ShapeDtypeStruct((B, S, H), jnp.float32)
    if prev is None:
        return pl.pallas_call(
            _ln_body,
            grid=grid,
            in_specs=data_specs,
            out_specs=out_spec,
            out_shape=out_shape,
        )(scale, concat, tok_p, pos, gamma, beta)

    def body(prev_ref, *refs):
        _ln_body(*refs)

    return pl.pallas_call(
        body,
        grid=grid,
        in_specs=[pl.BlockSpec(memory_space=pltpu.MemorySpace.HBM)] + data_specs,
        out_specs=out_spec,
        out_shape=out_shape,
        input_output_aliases={0: 0},
    )(prev, scale, concat, tok_p, pos, gamma, beta)


def kernel(concat_embeddings, concat_type, pos_table, tok_table, ln_gamma, ln_beta):
    idx = concat_type.reshape(_P, _PROWS).astype(jnp.int32)
    absmax = jnp.maximum(jnp.max(jnp.abs(tok_table)), 1e-30)
    q = jnp.round(tok_table * (127.0 / absmax)).astype(jnp.int8)
    table_packed = lax.bitcast_convert_type(
        q.reshape(H, 4, _W).transpose(0, 2, 1), jnp.int32
    )
    scale = (absmax / 127.0).reshape(1)
    gamma = ln_gamma.reshape(1, H)
    beta = ln_beta.reshape(1, H)
    toks = [
        _sc_gather(table_packed, idx[p]).reshape(_PB, S, _W) for p in range(_P)
    ]
    out = None
    for p in range(_P):
        out = _tc_piece(
            out, scale, concat_embeddings, toks[p], pos_table, gamma, beta, p
        )
    return out
